# Initial kernel scaffold; baseline (speedup 1.0000x reference)
import jax
import jax.numpy as jnp
from jax.experimental import pallas as pl


def kernel(chld_prt_tokens, types, positions, embed_weight, pos_weight):
    raise NotImplementedError("write your pallas kernel here")


# speedup vs baseline: 1.6776x; 1.6776x over previous
import functools

import jax
import jax.numpy as jnp
from jax import lax
from jax.experimental import pallas as pl
from jax.experimental.pallas import tpu as pltpu
from jax.experimental.pallas import tpu_sc as plsc

EMB = 64
PAD = 0
NEG = -99999999.0

NC, NS = 2, 16
NW = NC * NS

CH = 640
SUB = 128
NSUB = CH // SUB


def _sc_gather(idx1d, table, total_rows):
    rpw = total_rows // NW
    nchunk = rpw // CH
    mesh = plsc.VectorSubcoreMesh(
        core_axis_name="c", subcore_axis_name="s",
        num_cores=NC, num_subcores=NS)

    @functools.partial(
        pl.kernel,
        out_type=jax.ShapeDtypeStruct((total_rows, EMB), jnp.float32),
        mesh=mesh,
        scratch_types=[
            pltpu.VMEM((CH,), jnp.int32),
            pltpu.VMEM((CH, EMB), jnp.float32),
            pltpu.SemaphoreType.DMA,
        ],
        compiler_params=pltpu.CompilerParams(use_tc_tiling_on_sc=False),
    )
    def k(idx_hbm, table_hbm, out_hbm, idx_v, rows_v, sem):
        wid = lax.axis_index("s") * NC + lax.axis_index("c")
        row0 = wid * rpw

        def chunk_body(i, carry):
            off = row0 + i * CH
            pltpu.sync_copy(idx_hbm.at[pl.ds(off, CH)], idx_v)
            copies = []
            for j in range(NSUB):
                copies.append(pltpu.make_async_copy(
                    table_hbm.at[idx_v.at[pl.ds(j * SUB, SUB)]],
                    rows_v.at[pl.ds(j * SUB, SUB)],
                    sem))
            for c in copies:
                c.start()
            for c in copies:
                c.wait()
            pltpu.sync_copy(rows_v, out_hbm.at[pl.ds(off, CH)])
            return carry

        lax.fori_loop(0, nchunk, chunk_body, 0)

    return k(idx1d, table)


def _tc_dense(gathered, tok, pos, pos_weight, bsz_num):
    S = tok.shape[1]
    BLK = 256
    grid = (bsz_num // BLK,)

    def body(g_ref, tok_ref, pos_ref, pw_ref, out_ref):
        e = g_ref[...]
        p = pos_ref[...]
        t = tok_ref[...]
        scores = jnp.zeros((BLK, S), jnp.float32)
        for kk in range(pw_ref.shape[0]):
            row = pw_ref[kk:kk + 1, :].reshape(1, 1, EMB)
            sk = jnp.sum(e * row, axis=2)
            scores = scores + jnp.where(p == kk, sk, 0.0)
        scores = jnp.where(t == PAD, NEG, scores)
        m = jnp.max(scores, axis=1, keepdims=True)
        w = jnp.exp(scores - m)
        w = w / jnp.sum(w, axis=1, keepdims=True)
        acc = jnp.zeros((BLK, EMB), jnp.float32)
        for s in range(S):
            acc = acc + w[:, s:s + 1] * g_ref[:, s, :]
        out_ref[...] = acc

    return pl.pallas_call(
        body,
        grid=grid,
        in_specs=[
            pl.BlockSpec((BLK, S, EMB), lambda i: (i, 0, 0)),
            pl.BlockSpec((BLK, S), lambda i: (i, 0)),
            pl.BlockSpec((BLK, S), lambda i: (i, 0)),
            pl.BlockSpec((7, EMB), lambda i: (0, 0)),
        ],
        out_specs=pl.BlockSpec((BLK, EMB), lambda i: (i, 0)),
        out_shape=jax.ShapeDtypeStruct((bsz_num, EMB), jnp.float32),
    )(gathered, tok, pos, pos_weight)


def kernel(chld_prt_tokens, types, positions, embed_weight, pos_weight):
    bsz, num, seq_len = chld_prt_tokens.shape
    bn = bsz * num
    total = bn * seq_len
    tok2d = chld_prt_tokens.reshape(bn, seq_len)
    pos2d = positions.reshape(bn, seq_len)
    idx1d = chld_prt_tokens.reshape(total)

    gathered = _sc_gather(idx1d, embed_weight, total)
    gathered = gathered.reshape(bn, seq_len, EMB)
    res = _tc_dense(gathered, tok2d, pos2d, pos_weight, bn)
    return res.reshape(bsz, num, EMB)


# speedup vs baseline: 2.0151x; 1.2012x over previous
import functools

import jax
import jax.numpy as jnp
from jax import lax
from jax.experimental import pallas as pl
from jax.experimental.pallas import tpu as pltpu
from jax.experimental.pallas import tpu_sc as plsc

EMB = 64
PAD = 0
NEG = -99999999.0

NC, NS = 2, 16
NW = NC * NS

CH = 640
SUB = 128
NSUB = CH // SUB


def _sc_gather(idx1d, table, total_rows):
    rpw = total_rows // NW
    nchunk = rpw // CH
    mesh = plsc.VectorSubcoreMesh(
        core_axis_name="c", subcore_axis_name="s",
        num_cores=NC, num_subcores=NS)

    @functools.partial(
        pl.kernel,
        out_type=jax.ShapeDtypeStruct((total_rows, EMB), jnp.float32),
        mesh=mesh,
        scratch_types=[
            pltpu.VMEM((2, CH), jnp.int32),
            pltpu.VMEM((2, CH, EMB), jnp.float32),
            pltpu.SemaphoreType.DMA,
            pltpu.SemaphoreType.DMA,
            pltpu.SemaphoreType.DMA,
        ],
        compiler_params=pltpu.CompilerParams(use_tc_tiling_on_sc=False),
    )
    def k(idx_hbm, table_hbm, out_hbm, idx_v, rows_v, isem, gsem, osem):
        wid = lax.axis_index("s") * NC + lax.axis_index("c")
        row0 = wid * rpw

        def idx_copy(i, b):
            return pltpu.make_async_copy(
                idx_hbm.at[pl.ds(row0 + i * CH, CH)], idx_v.at[b], isem)

        def gather_copies(b):
            return [pltpu.make_async_copy(
                table_hbm.at[idx_v.at[b, pl.ds(j * SUB, SUB)]],
                rows_v.at[b, pl.ds(j * SUB, SUB)], gsem)
                for j in range(NSUB)]

        def out_copy(i, b):
            return pltpu.make_async_copy(
                rows_v.at[b], out_hbm.at[pl.ds(row0 + i * CH, CH)], osem)

        idx_copy(0, 0).start()
        for i in range(nchunk):
            b = i % 2
            if i > 0:
                for c in gather_copies(1 - b):
                    c.wait()
                out_copy(i - 1, 1 - b).start()
            idx_copy(i, b).wait()
            if i >= 2:
                out_copy(i - 2, b).wait()
            for c in gather_copies(b):
                c.start()
            if i + 1 < nchunk:
                idx_copy(i + 1, 1 - b).start()
        bl = (nchunk - 1) % 2
        for c in gather_copies(bl):
            c.wait()
        out_copy(nchunk - 1, bl).start()
        out_copy(nchunk - 2, 1 - bl).wait()
        out_copy(nchunk - 1, bl).wait()

    return k(idx1d, table)


def _tc_dense(gathered, tok, pos, pos_weight, bsz_num):
    S = tok.shape[1]
    BLK = 256
    grid = (bsz_num // BLK,)
    pwt = jnp.pad(pos_weight, ((0, 1), (0, 0))).T

    def body(g_ref, tok_ref, pos_ref, pwt_ref, out_ref):
        e = g_ref[...]
        e2 = e.reshape(BLK * S, EMB)
        s7 = jnp.dot(e2, pwt_ref[...], preferred_element_type=jnp.float32,
                     precision=lax.Precision.HIGHEST)
        s73 = s7.reshape(BLK, S, 8)
        p3 = pos_ref[...].reshape(BLK, S, 1)
        t3 = tok_ref[...].reshape(BLK, S, 1)
        i3 = lax.broadcasted_iota(jnp.int32, (BLK, S, 8), 2)
        sc = jnp.sum(jnp.where(i3 == p3, s73, 0.0), axis=2, keepdims=True)
        sc = jnp.where(t3 == PAD, NEG, sc)
        m = jnp.max(sc, axis=1, keepdims=True)
        w = jnp.exp(sc - m)
        w = w / jnp.sum(w, axis=1, keepdims=True)
        out_ref[...] = jnp.sum(w * e, axis=1)

    return pl.pallas_call(
        body,
        grid=grid,
        in_specs=[
            pl.BlockSpec((BLK, S, EMB), lambda i: (i, 0, 0)),
            pl.BlockSpec((BLK, S), lambda i: (i, 0)),
            pl.BlockSpec((BLK, S), lambda i: (i, 0)),
            pl.BlockSpec((EMB, 8), lambda i: (0, 0)),
        ],
        out_specs=pl.BlockSpec((BLK, EMB), lambda i: (i, 0)),
        out_shape=jax.ShapeDtypeStruct((bsz_num, EMB), jnp.float32),
    )(gathered, tok, pos, pwt)


def kernel(chld_prt_tokens, types, positions, embed_weight, pos_weight):
    bsz, num, seq_len = chld_prt_tokens.shape
    bn = bsz * num
    total = bn * seq_len
    tok2d = chld_prt_tokens.reshape(bn, seq_len)
    pos2d = positions.reshape(bn, seq_len)
    idx1d = chld_prt_tokens.reshape(total)

    gathered = _sc_gather(idx1d, embed_weight, total)
    gathered = gathered.reshape(bn, seq_len, EMB)
    res = _tc_dense(gathered, tok2d, pos2d, pos_weight, bn)
    return res.reshape(bsz, num, EMB)


# speedup vs baseline: 2.0351x; 1.0099x over previous
import functools

import jax
import jax.numpy as jnp
from jax import lax
from jax.experimental import pallas as pl
from jax.experimental.pallas import tpu as pltpu
from jax.experimental.pallas import tpu_sc as plsc

EMB = 64
PAD = 0
NEG = -99999999.0

NC, NS = 2, 16
NW = NC * NS

CH = 128
NBUF = 4
WIDE, NARROW = 104, 96


def _sc_gather_pairs(idx2d, table2, total_rows):
    mesh = plsc.VectorSubcoreMesh(
        core_axis_name="c", subcore_axis_name="s",
        num_cores=NC, num_subcores=NS)

    @functools.partial(
        pl.kernel,
        out_type=jax.ShapeDtypeStruct((total_rows, 2 * EMB), jnp.float32),
        mesh=mesh,
        scratch_types=[
            pltpu.VMEM((WIDE, CH), jnp.int32),
            pltpu.VMEM((NBUF, CH, 2 * EMB), jnp.float32),
            [pltpu.SemaphoreType.DMA] * NBUF,
            [pltpu.SemaphoreType.DMA] * NBUF,
        ],
    )
    def k(idx_hbm, table_hbm, out_hbm, idx_v, rows_v, gsems, osems):
        wid = lax.axis_index("s") * NC + lax.axis_index("c")
        is_wide = (wid % 2) == 0
        base_row = (wid // 2) * (WIDE + NARROW) + jnp.where(is_wide, 0, WIDE)
        nch = jnp.where(is_wide, WIDE, NARROW)
        out0 = base_row * CH

        pltpu.sync_copy(idx_hbm.at[pl.ds(base_row, NARROW)],
                        idx_v.at[pl.ds(0, NARROW)])

        @pl.when(is_wide)
        def _():
            pltpu.sync_copy(idx_hbm.at[pl.ds(base_row + NARROW, WIDE - NARROW)],
                            idx_v.at[pl.ds(NARROW, WIDE - NARROW)])

        def g_copy(c, q):
            return pltpu.make_async_copy(
                table_hbm.at[idx_v.at[c]], rows_v.at[q], gsems[q])

        def o_copy(c, q):
            return pltpu.make_async_copy(
                rows_v.at[q], out_hbm.at[pl.ds(out0 + c * CH, CH)], osems[q])

        for q in range(NBUF):
            g_copy(q, q).start()

        def step(i, carry):
            c0 = i * NBUF
            for q in range(NBUF):
                g_copy(c0 + q, q).wait()
                o_copy(c0 + q, q).start()

                @pl.when(c0 + NBUF + q < nch)
                def _(q=q):
                    o_copy(c0 + q, q).wait()
                    g_copy(c0 + NBUF + q, q).start()
            return carry

        lax.fori_loop(0, nch // NBUF, step, 0)
        for q in range(NBUF):
            o_copy(0, q).wait()

    return k(idx2d, table2)


def _tc_dense(gpairs, tok, pos, pos_weight, bsz_num):
    S = tok.shape[1]
    BLK = 256
    grid = (bsz_num // BLK,)
    pwt = jnp.pad(pos_weight, ((0, 1), (0, 0))).T

    def body(g_ref, tok_ref, pos_ref, pwt_ref, out_ref):
        ep = g_ref[...]
        t3 = tok_ref[...].reshape(BLK, S, 1)
        lo = ep[:, :EMB].reshape(BLK, S, EMB)
        hi = ep[:, EMB:].reshape(BLK, S, EMB)
        e = jnp.where((t3 & 1) == 1, hi, lo)
        e2 = e.reshape(BLK * S, EMB)
        s7 = jnp.dot(e2, pwt_ref[...], preferred_element_type=jnp.float32,
                     precision=lax.Precision.HIGHEST)
        s73 = s7.reshape(BLK, S, 8)
        p3 = pos_ref[...].reshape(BLK, S, 1)
        i3 = lax.broadcasted_iota(jnp.int32, (BLK, S, 8), 2)
        sc = jnp.sum(jnp.where(i3 == p3, s73, 0.0), axis=2, keepdims=True)
        sc = jnp.where(t3 == PAD, NEG, sc)
        m = jnp.max(sc, axis=1, keepdims=True)
        w = jnp.exp(sc - m)
        w = w / jnp.sum(w, axis=1, keepdims=True)
        out_ref[...] = jnp.sum(w * e, axis=1)

    return pl.pallas_call(
        body,
        grid=grid,
        in_specs=[
            pl.BlockSpec((BLK * S, 2 * EMB), lambda i: (i, 0)),
            pl.BlockSpec((BLK, S), lambda i: (i, 0)),
            pl.BlockSpec((BLK, S), lambda i: (i, 0)),
            pl.BlockSpec((EMB, 8), lambda i: (0, 0)),
        ],
        out_specs=pl.BlockSpec((BLK, EMB), lambda i: (i, 0)),
        out_shape=jax.ShapeDtypeStruct((bsz_num, EMB), jnp.float32),
    )(gpairs, tok, pos, pwt)


def kernel(chld_prt_tokens, types, positions, embed_weight, pos_weight):
    bsz, num, seq_len = chld_prt_tokens.shape
    bn = bsz * num
    total = bn * seq_len
    tok2d = chld_prt_tokens.reshape(bn, seq_len)
    pos2d = positions.reshape(bn, seq_len)
    idx2d = (chld_prt_tokens >> 1).reshape(total // 128, 128)
    table2 = embed_weight.reshape(embed_weight.shape[0] // 2, 2 * EMB)

    gpairs = _sc_gather_pairs(idx2d, table2, total)
    res = _tc_dense(gpairs, tok2d, pos2d, pos_weight, bn)
    return res.reshape(bsz, num, EMB)


# speedup vs baseline: 2.4294x; 1.1938x over previous
import functools

import jax
import jax.numpy as jnp
from jax import lax
from jax.experimental import pallas as pl
from jax.experimental.pallas import tpu as pltpu
from jax.experimental.pallas import tpu_sc as plsc

EMB = 64
PAD = 0
NEG = -99999999.0

NC, NS = 2, 16
NW = NC * NS

CH = 128
NBUF = 4
WIDE, NARROW = 104, 96


def _sc_gather_pairs(idx2d, table2, total_rows):
    mesh = plsc.VectorSubcoreMesh(
        core_axis_name="c", subcore_axis_name="s",
        num_cores=NC, num_subcores=NS)

    @functools.partial(
        pl.kernel,
        out_type=jax.ShapeDtypeStruct((total_rows, 2 * EMB), jnp.float32),
        mesh=mesh,
        scratch_types=[
            pltpu.VMEM((WIDE, CH), jnp.int32),
            pltpu.VMEM((NBUF, CH, 2 * EMB), jnp.float32),
            [pltpu.SemaphoreType.DMA] * NBUF,
            [pltpu.SemaphoreType.DMA] * NBUF,
        ],
    )
    def k(idx_hbm, table_hbm, out_hbm, idx_v, rows_v, gsems, osems):
        wid = lax.axis_index("s") * NC + lax.axis_index("c")
        is_wide = (wid % 2) == 0
        base_row = (wid // 2) * (WIDE + NARROW) + jnp.where(is_wide, 0, WIDE)
        nch = jnp.where(is_wide, WIDE, NARROW)
        out0 = base_row * CH

        pltpu.sync_copy(idx_hbm.at[pl.ds(base_row, NARROW)],
                        idx_v.at[pl.ds(0, NARROW)])

        @pl.when(is_wide)
        def _():
            pltpu.sync_copy(idx_hbm.at[pl.ds(base_row + NARROW, WIDE - NARROW)],
                            idx_v.at[pl.ds(NARROW, WIDE - NARROW)])

        def g_copy(c, q):
            return pltpu.make_async_copy(
                table_hbm.at[idx_v.at[c]], rows_v.at[q], gsems[q])

        def o_copy(c, q):
            return pltpu.make_async_copy(
                rows_v.at[q], out_hbm.at[pl.ds(out0 + c * CH, CH)], osems[q])

        for q in range(NBUF):
            g_copy(q, q).start()

        def step(i, carry):
            c0 = i * NBUF
            for q in range(NBUF):
                g_copy(c0 + q, q).wait()
                o_copy(c0 + q, q).start()

                @pl.when(c0 + NBUF + q < nch)
                def _(q=q):
                    o_copy(c0 + q, q).wait()
                    g_copy(c0 + NBUF + q, q).start()
            return carry

        lax.fori_loop(0, nch // NBUF, step, 0)
        for q in range(NBUF):
            o_copy(0, q).wait()

    return k(idx2d, table2)


def _tc_transpose_pad(table_t, vocab):
    C = 2048
    grid = (pl.cdiv(vocab, C),)

    def body(t_ref, out_ref):
        x = t_ref[...]
        xt = jnp.swapaxes(x, 0, 1)
        out_ref[...] = jnp.concatenate(
            [xt, jnp.zeros((C, 2 * EMB - xt.shape[1]), jnp.float32)], axis=1)

    return pl.pallas_call(
        body,
        grid=grid,
        in_specs=[pl.BlockSpec((EMB, C), lambda i: (0, i))],
        out_specs=pl.BlockSpec((C, 2 * EMB), lambda i: (i, 0)),
        out_shape=jax.ShapeDtypeStruct((vocab, 2 * EMB), jnp.float32),
    )(table_t)


def _tc_dense(gpairs, tok, pos, pos_weight, bsz_num):
    S = tok.shape[1]
    BLK = 256
    grid = (bsz_num // BLK,)
    pwt = jnp.pad(pos_weight, ((0, 1), (0, 0))).T

    def body(g_ref, tok_ref, pos_ref, pwt_ref, out_ref):
        ep = g_ref[...]
        t3 = tok_ref[...].reshape(BLK, S, 1)
        e = ep[:, :EMB].reshape(BLK, S, EMB)
        e2 = e.reshape(BLK * S, EMB)
        s7 = jnp.dot(e2, pwt_ref[...], preferred_element_type=jnp.float32,
                     precision=lax.Precision.HIGHEST)
        s73 = s7.reshape(BLK, S, 8)
        p3 = pos_ref[...].reshape(BLK, S, 1)
        i3 = lax.broadcasted_iota(jnp.int32, (BLK, S, 8), 2)
        sc = jnp.sum(jnp.where(i3 == p3, s73, 0.0), axis=2, keepdims=True)
        sc = jnp.where(t3 == PAD, NEG, sc)
        m = jnp.max(sc, axis=1, keepdims=True)
        w = jnp.exp(sc - m)
        w = w / jnp.sum(w, axis=1, keepdims=True)
        out_ref[...] = jnp.sum(w * e, axis=1)

    return pl.pallas_call(
        body,
        grid=grid,
        in_specs=[
            pl.BlockSpec((BLK * S, 2 * EMB), lambda i: (i, 0)),
            pl.BlockSpec((BLK, S), lambda i: (i, 0)),
            pl.BlockSpec((BLK, S), lambda i: (i, 0)),
            pl.BlockSpec((EMB, 8), lambda i: (0, 0)),
        ],
        out_specs=pl.BlockSpec((BLK, EMB), lambda i: (i, 0)),
        out_shape=jax.ShapeDtypeStruct((bsz_num, EMB), jnp.float32),
    )(gpairs, tok, pos, pwt)


def kernel(chld_prt_tokens, types, positions, embed_weight, pos_weight):
    bsz, num, seq_len = chld_prt_tokens.shape
    bn = bsz * num
    total = bn * seq_len
    tok2d = chld_prt_tokens.reshape(bn, seq_len)
    pos2d = positions.reshape(bn, seq_len)
    idx2d = chld_prt_tokens.reshape(total // 128, 128)
    table2 = _tc_transpose_pad(embed_weight.T, embed_weight.shape[0])

    gpairs = _sc_gather_pairs(idx2d, table2, total)
    res = _tc_dense(gpairs, tok2d, pos2d, pos_weight, bn)
    return res.reshape(bsz, num, EMB)


# speedup vs baseline: 2.8140x; 1.1583x over previous
import functools

import jax
import jax.numpy as jnp
from jax import lax
from jax.experimental import pallas as pl
from jax.experimental.pallas import tpu as pltpu
from jax.experimental.pallas import tpu_sc as plsc

EMB = 64
PAD = 0
NEG = -99999999.0

NC, NS = 2, 16
NW = NC * NS

CH = 128
NBUF = 4


def _sc_gather_rows(idx2d, table2, total_rows, big, small):
    nrow = total_rows // CH
    assert 3 * big + small == nrow // 8 and big % 8 == 0 and small % 8 == 0
    mesh = plsc.VectorSubcoreMesh(
        core_axis_name="c", subcore_axis_name="s",
        num_cores=NC, num_subcores=NS)

    @functools.partial(
        pl.kernel,
        out_type=jax.ShapeDtypeStruct((total_rows, 2 * EMB), jnp.float32),
        mesh=mesh,
        scratch_types=[
            pltpu.VMEM((big, CH), jnp.int32),
            pltpu.VMEM((NBUF, CH, 2 * EMB), jnp.float32),
            [pltpu.SemaphoreType.DMA] * NBUF,
            [pltpu.SemaphoreType.DMA] * NBUF,
        ],
    )
    def k(idx_hbm, table_hbm, out_hbm, idx_v, rows_v, gsems, osems):
        wid = lax.axis_index("s") * NC + lax.axis_index("c")
        m = wid % 4
        is_big = m < 3
        base_row = (wid // 4) * (3 * big + small) + m * big
        nch = jnp.where(is_big, big, small)
        out0 = base_row * CH

        pltpu.sync_copy(idx_hbm.at[pl.ds(base_row, small)],
                        idx_v.at[pl.ds(0, small)])

        @pl.when(is_big)
        def _():
            pltpu.sync_copy(idx_hbm.at[pl.ds(base_row + small, big - small)],
                            idx_v.at[pl.ds(small, big - small)])

        def g_copy(c, q):
            return pltpu.make_async_copy(
                table_hbm.at[idx_v.at[c]], rows_v.at[q], gsems[q])

        def o_copy(c, q):
            return pltpu.make_async_copy(
                rows_v.at[q], out_hbm.at[pl.ds(out0 + c * CH, CH)], osems[q])

        for q in range(NBUF):
            g_copy(q, q).start()

        def step(i, carry):
            c0 = i * NBUF
            for q in range(NBUF):
                g_copy(c0 + q, q).wait()
                o_copy(c0 + q, q).start()

                @pl.when(c0 + NBUF + q < nch)
                def _(q=q):
                    o_copy(c0 + q, q).wait()
                    g_copy(c0 + NBUF + q, q).start()
            return carry

        lax.fori_loop(0, nch // NBUF, step, 0)
        for q in range(NBUF):
            o_copy(0, q).wait()

    return k(idx2d, table2)


BLK = 512


def _tc_transpose_pad(table_t, vocab):
    C = 4096
    grid = (pl.cdiv(vocab, C),)

    def body(t_ref, out_ref):
        x = t_ref[...]
        xt = jnp.swapaxes(x, 0, 1)
        out_ref[...] = jnp.concatenate(
            [xt, jnp.zeros((C, EMB), jnp.float32)], axis=1)

    return pl.pallas_call(
        body,
        grid=grid,
        in_specs=[pl.BlockSpec((EMB, C), lambda i: (0, i))],
        out_specs=pl.BlockSpec((C, 2 * EMB), lambda i: (i, 0)),
        out_shape=jax.ShapeDtypeStruct((vocab, 2 * EMB), jnp.float32),
    )(table_t)


def _tc_dense(gpairs, tok, pos, pos_weight, bsz_num):
    S = tok.shape[1]
    grid = (bsz_num // BLK,)
    pwt = jnp.pad(pos_weight, ((0, 1), (0, 0))).T

    def body(g_ref, tok_ref, pos_ref, pwt_ref, out_ref):
        ep = g_ref[...]
        t3 = tok_ref[...].reshape(BLK, S, 1)
        e = ep[:, :EMB].reshape(BLK, S, EMB)
        e2 = e.reshape(BLK * S, EMB)
        s7 = jnp.dot(e2, pwt_ref[...], preferred_element_type=jnp.float32,
                     precision=lax.Precision.HIGHEST)
        s73 = s7.reshape(BLK, S, 8)
        p3 = pos_ref[...].reshape(BLK, S, 1)
        i3 = lax.broadcasted_iota(jnp.int32, (BLK, S, 8), 2)
        sc = jnp.sum(jnp.where(i3 == p3, s73, 0.0), axis=2, keepdims=True)
        sc = jnp.where(t3 == PAD, NEG, sc)
        m = jnp.max(sc, axis=1, keepdims=True)
        w = jnp.exp(sc - m)
        w = w / jnp.sum(w, axis=1, keepdims=True)
        out_ref[...] = jnp.sum(w * e, axis=1)

    return pl.pallas_call(
        body,
        grid=grid,
        in_specs=[
            pl.BlockSpec((BLK * S, 2 * EMB), lambda i: (i, 0)),
            pl.BlockSpec((BLK, S), lambda i: (i, 0)),
            pl.BlockSpec((BLK, S), lambda i: (i, 0)),
            pl.BlockSpec((EMB, 8), lambda i: (0, 0)),
        ],
        out_specs=pl.BlockSpec((BLK, EMB), lambda i: (i, 0)),
        out_shape=jax.ShapeDtypeStruct((bsz_num, EMB), jnp.float32),
    )(gpairs, tok, pos, pwt)


def kernel(chld_prt_tokens, types, positions, embed_weight, pos_weight):
    bsz, num, seq_len = chld_prt_tokens.shape
    bn = bsz * num
    total = bn * seq_len
    tok2d = chld_prt_tokens.reshape(bn, seq_len)
    pos2d = positions.reshape(bn, seq_len)
    idx2d = chld_prt_tokens.reshape(total // 128, 128)
    table2 = _tc_transpose_pad(embed_weight.T, embed_weight.shape[0])

    hr = total // 2
    hn = bn // 2
    hi2 = idx2d.shape[0] // 2
    g1 = _sc_gather_rows(idx2d[:hi2], table2, hr, 56, 32)
    g2 = _sc_gather_rows(idx2d[hi2:], table2, hr, 56, 32)
    r1 = _tc_dense(g1, tok2d[:hn], pos2d[:hn], pos_weight, hn)
    r2 = _tc_dense(g2, tok2d[hn:], pos2d[hn:], pos_weight, hn)
    res = jnp.concatenate([r1, r2], axis=0)
    return res.reshape(bsz, num, EMB)
